# baseline (speedup 1.0000x reference)
import functools

import jax
import jax.numpy as jnp
from jax import lax
from jax.experimental import pallas as pl
from jax.experimental.pallas import tpu as pltpu
from jax.experimental.pallas import tpu_sc as plsc

N_NODES = 10000
FEAT = 128
N_GRAPHS = 64

NC = 2
NT = 16
NW = NC * NT
NP = 10112
SLICE = NP // NT

BLK = 1000
GRID = N_NODES // BLK


def _sc_mesh():
    return plsc.VectorSubcoreMesh(core_axis_name="c", subcore_axis_name="s")


def _deg_call(coli, ones128, zeros128):
    chunks = coli.shape[0]
    rpt = chunks // NW

    @functools.partial(
        pl.kernel,
        out_type=jax.ShapeDtypeStruct((NC, NP, FEAT), jnp.float32),
        mesh=_sc_mesh(),
        scratch_types=[
            pltpu.VMEM((rpt, 128), jnp.int32),
            pltpu.VMEM((128, FEAT), jnp.float32),
            pltpu.VMEM_SHARED((NP, FEAT), jnp.float32),
            pltpu.SemaphoreType.DMA,
        ],
    )
    def deg_k(coli_hbm, ones_hbm, zeros_hbm, out_hbm, colv, onesv, deg_sh,
              sem):
        c = lax.axis_index("c")
        s = lax.axis_index("s")
        wid = c * NT + s
        pltpu.sync_copy(zeros_hbm, deg_sh.at[pl.ds(s * SLICE, SLICE)])
        pltpu.sync_copy(coli_hbm.at[pl.ds(wid * rpt, rpt)], colv)
        pltpu.sync_copy(ones_hbm, onesv)
        plsc.subcore_barrier()

        def wait_one():
            pltpu.make_async_copy(ones_hbm, onesv, sem).wait()

        pltpu.async_copy(onesv, deg_sh.at[colv.at[0]], sem, add=True)

        def body(j, carry):
            pltpu.async_copy(onesv, deg_sh.at[colv.at[j]], sem, add=True)
            wait_one()
            return carry

        lax.fori_loop(1, rpt, body, 0)
        wait_one()
        plsc.subcore_barrier()
        pltpu.sync_copy(deg_sh.at[pl.ds(s * SLICE, SLICE)],
                        out_hbm.at[c, pl.ds(s * SLICE, SLICE)])

    return deg_k(coli, ones128, zeros128)


def _agg_call(hs, rowi, coli, zeros128):
    chunks = rowi.shape[0]
    rpt = chunks // NW

    @functools.partial(
        pl.kernel,
        out_type=jax.ShapeDtypeStruct((NC, NP, FEAT), jnp.float32),
        mesh=_sc_mesh(),
        scratch_types=[
            pltpu.VMEM((16, 128), jnp.int32),
            pltpu.VMEM((16, 128), jnp.int32),
            pltpu.VMEM((128, FEAT), jnp.float32),
            pltpu.VMEM((128, FEAT), jnp.float32),
            pltpu.VMEM_SHARED((NP, FEAT), jnp.float32),
            pltpu.SemaphoreType.DMA,
            pltpu.SemaphoreType.DMA,
            pltpu.SemaphoreType.DMA,
            pltpu.SemaphoreType.DMA,
        ],
    )
    def agg_k(hs_hbm, rowi_hbm, coli_hbm, zeros_hbm, out_hbm,
              rowv, colv, gbuf0, gbuf1, acc_sh, gs0, gs1, ss0, ss1):
        c = lax.axis_index("c")
        s = lax.axis_index("s")
        wid = c * NT + s
        seg = 16
        nseg = rpt // seg
        pltpu.sync_copy(zeros_hbm, acc_sh.at[pl.ds(s * SLICE, SLICE)])
        plsc.subcore_barrier()

        def wait_g(sem_, buf):
            pltpu.make_async_copy(hs_hbm.at[rowv.at[0]], buf, sem_).wait()

        def wait_s(sem_, buf):
            pltpu.make_async_copy(hs_hbm.at[rowv.at[0]], buf, sem_).wait()

        def seg_body(g, carry):
            base = wid * rpt + g * seg
            pltpu.sync_copy(rowi_hbm.at[pl.ds(base, seg)], rowv)
            pltpu.sync_copy(coli_hbm.at[pl.ds(base, seg)], colv)
            pltpu.async_copy(hs_hbm.at[rowv.at[0]], gbuf0, gs0)
            pltpu.async_copy(hs_hbm.at[rowv.at[1]], gbuf1, gs1)

            def body(t, c2):
                j0 = t * 2
                wait_g(gs0, gbuf0)
                pltpu.async_copy(gbuf0, acc_sh.at[colv.at[j0]], ss0,
                                 add=True)
                wait_g(gs1, gbuf1)
                pltpu.async_copy(gbuf1, acc_sh.at[colv.at[j0 + 1]], ss1,
                                 add=True)

                @pl.when(t + 1 < seg // 2)
                def _():
                    wait_s(ss0, gbuf0)
                    pltpu.async_copy(hs_hbm.at[rowv.at[j0 + 2]], gbuf0, gs0)
                    wait_s(ss1, gbuf1)
                    pltpu.async_copy(hs_hbm.at[rowv.at[j0 + 3]], gbuf1, gs1)

                return c2

            lax.fori_loop(0, seg // 2, body, 0)
            wait_s(ss0, gbuf0)
            wait_s(ss1, gbuf1)
            return carry

        lax.fori_loop(0, nseg, seg_body, 0)
        plsc.subcore_barrier()
        pltpu.sync_copy(acc_sh.at[pl.ds(s * SLICE, SLICE)],
                        out_hbm.at[c, pl.ds(s * SLICE, SLICE)])

    return agg_k(hs, rowi, coli, zeros128)


def _first_tc(degp, x, W1):
    def body(degp_ref, x_ref, w_ref, hs_ref, disb_ref):
        deg = degp_ref[0, :, 0:1] + degp_ref[1, :, 0:1] + 1.0
        dis = lax.rsqrt(deg)
        h = jnp.dot(x_ref[...], w_ref[...], preferred_element_type=jnp.float32)
        hs_ref[...] = h * dis
        disb_ref[...] = jnp.broadcast_to(dis, (BLK, FEAT))

    return pl.pallas_call(
        body,
        grid=(GRID,),
        in_specs=[
            pl.BlockSpec((NC, BLK, FEAT), lambda i: (0, i, 0)),
            pl.BlockSpec((BLK, FEAT), lambda i: (i, 0)),
            pl.BlockSpec((FEAT, FEAT), lambda i: (0, 0)),
        ],
        out_specs=[
            pl.BlockSpec((BLK, FEAT), lambda i: (i, 0)),
            pl.BlockSpec((BLK, FEAT), lambda i: (i, 0)),
        ],
        out_shape=[
            jax.ShapeDtypeStruct((N_NODES, FEAT), jnp.float32),
            jax.ShapeDtypeStruct((N_NODES, FEAT), jnp.float32),
        ],
    )(degp, x, W1)


def _mid_tc(accp, hs, disb, b, W):
    def body(accp_ref, hs_ref, disb_ref, b_ref, w_ref, out_ref):
        agg = accp_ref[0] + accp_ref[1] + hs_ref[...]
        xn = jnp.maximum(disb_ref[...] * agg + b_ref[...], 0.0)
        h = jnp.dot(xn, w_ref[...], preferred_element_type=jnp.float32)
        out_ref[...] = h * disb_ref[...]

    return pl.pallas_call(
        body,
        grid=(GRID,),
        in_specs=[
            pl.BlockSpec((NC, BLK, FEAT), lambda i: (0, i, 0)),
            pl.BlockSpec((BLK, FEAT), lambda i: (i, 0)),
            pl.BlockSpec((BLK, FEAT), lambda i: (i, 0)),
            pl.BlockSpec((1, FEAT), lambda i: (0, 0)),
            pl.BlockSpec((FEAT, FEAT), lambda i: (0, 0)),
        ],
        out_specs=pl.BlockSpec((BLK, FEAT), lambda i: (i, 0)),
        out_shape=jax.ShapeDtypeStruct((N_NODES, FEAT), jnp.float32),
    )(accp, hs, disb, b, W)


def _final_tc(accp, hs, disb, b, batch3, Wf1, bf1, Wf2, bf2):
    h3 = Wf1.shape[1]
    nout = Wf2.shape[1]

    def body(accp_ref, hs_ref, disb_ref, b_ref, batch_ref,
             wf1_ref, bf1_ref, wf2_ref, bf2_ref, out_ref, pooled, counts):
        i = pl.program_id(0)

        @pl.when(i == 0)
        def _():
            pooled[...] = jnp.zeros_like(pooled)
            counts[...] = jnp.zeros_like(counts)

        agg = accp_ref[0] + accp_ref[1] + hs_ref[...]
        x3 = jnp.maximum(disb_ref[...] * agg + b_ref[...], 0.0)
        batch_row = batch_ref[0]
        giota = lax.broadcasted_iota(jnp.int32, (N_GRAPHS, BLK), 0)
        onehot_t = (giota == batch_row).astype(jnp.float32)
        pooled[...] += lax.dot_general(
            onehot_t, x3, (((1,), (0,)), ((), ())),
            preferred_element_type=jnp.float32)
        counts[...] += jnp.broadcast_to(
            jnp.sum(onehot_t, axis=1, keepdims=True), (N_GRAPHS, FEAT))

        @pl.when(i == GRID - 1)
        def _():
            mean = pooled[...] / jnp.maximum(counts[...], 1.0)
            hmid = jnp.dot(mean, wf1_ref[...],
                           preferred_element_type=jnp.float32) + bf1_ref[...]
            out_ref[...] = jnp.dot(hmid, wf2_ref[...],
                                   preferred_element_type=jnp.float32) + bf2_ref[...]

    return pl.pallas_call(
        body,
        grid=(GRID,),
        in_specs=[
            pl.BlockSpec((NC, BLK, FEAT), lambda i: (0, i, 0)),
            pl.BlockSpec((BLK, FEAT), lambda i: (i, 0)),
            pl.BlockSpec((BLK, FEAT), lambda i: (i, 0)),
            pl.BlockSpec((1, FEAT), lambda i: (0, 0)),
            pl.BlockSpec((1, 1, BLK), lambda i: (i, 0, 0)),
            pl.BlockSpec((FEAT, h3), lambda i: (0, 0)),
            pl.BlockSpec((1, h3), lambda i: (0, 0)),
            pl.BlockSpec((h3, nout), lambda i: (0, 0)),
            pl.BlockSpec((1, nout), lambda i: (0, 0)),
        ],
        out_specs=pl.BlockSpec((N_GRAPHS, nout), lambda i: (0, 0)),
        out_shape=jax.ShapeDtypeStruct((N_GRAPHS, nout), jnp.float32),
        scratch_shapes=[
            pltpu.VMEM((N_GRAPHS, FEAT), jnp.float32),
            pltpu.VMEM((N_GRAPHS, FEAT), jnp.float32),
        ],
    )(accp, hs, disb, b, batch3, Wf1, bf1, Wf2, bf2)


def kernel(x, edge_index, batch, W1, b1, W2, b2, Wf1, bf1, Wf2, bf2):
    row = edge_index[0]
    col = edge_index[1]
    n_edges = row.shape[0]
    quantum = NW * 128 * 8
    ep = ((n_edges + quantum - 1) // quantum) * quantum
    pad = ep - n_edges
    pad_iota = jnp.arange(pad, dtype=jnp.int32)
    rowp = jnp.concatenate(
        [row, pad_iota % N_NODES]).reshape(ep // 128, 128)
    colp = jnp.concatenate(
        [col, N_NODES + pad_iota % (NP - N_NODES)]).reshape(ep // 128, 128)

    ones128 = jnp.ones((128, FEAT), jnp.float32)
    zeros128 = jnp.zeros((SLICE, FEAT), jnp.float32)
    b1r = b1.reshape(1, FEAT)
    b2r = b2.reshape(1, FEAT)
    bf1r = bf1.reshape(1, -1)
    bf2r = bf2.reshape(1, -1)
    batch3 = batch.reshape(GRID, 1, BLK)

    degp = _deg_call(colp, ones128, zeros128)
    hs1, disb = _first_tc(degp, x, W1)
    a1 = _agg_call(hs1, rowp, colp, zeros128)
    hs2 = _mid_tc(a1, hs1, disb, b1r, W2)
    a2 = _agg_call(hs2, rowp, colp, zeros128)
    return _final_tc(a2, hs2, disb, b2r, batch3, Wf1, bf1r, Wf2, bf2r)


# speedup vs baseline: 1.0715x; 1.0715x over previous
import functools

import jax
import jax.numpy as jnp
from jax import lax
from jax.experimental import pallas as pl
from jax.experimental.pallas import tpu as pltpu
from jax.experimental.pallas import tpu_sc as plsc

N_NODES = 10000
FEAT = 128
N_GRAPHS = 64

NC = 2
NT = 16
NW = NC * NT
NP = 10112
SLICE = NP // NT

BLK = 1000
GRID = N_NODES // BLK


def _sc_mesh():
    return plsc.VectorSubcoreMesh(core_axis_name="c", subcore_axis_name="s")


def _deg_call(coli, ones128, zeros128):
    chunks = coli.shape[0]
    rpt = chunks // NW

    @functools.partial(
        pl.kernel,
        out_type=jax.ShapeDtypeStruct((NC, NP, FEAT), jnp.float32),
        mesh=_sc_mesh(),
        scratch_types=[
            pltpu.VMEM((rpt, 128), jnp.int32),
            pltpu.VMEM((128, FEAT), jnp.float32),
            pltpu.VMEM_SHARED((NP, FEAT), jnp.float32),
            pltpu.SemaphoreType.DMA,
        ],
    )
    def deg_k(coli_hbm, ones_hbm, zeros_hbm, out_hbm, colv, onesv, deg_sh,
              sem):
        c = lax.axis_index("c")
        s = lax.axis_index("s")
        wid = c * NT + s
        pltpu.sync_copy(zeros_hbm, deg_sh.at[pl.ds(s * SLICE, SLICE)])
        pltpu.sync_copy(coli_hbm.at[pl.ds(wid * rpt, rpt)], colv)
        pltpu.sync_copy(ones_hbm, onesv)
        plsc.subcore_barrier()

        def wait_one():
            pltpu.make_async_copy(ones_hbm, onesv, sem).wait()

        pltpu.async_copy(onesv, deg_sh.at[colv.at[0]], sem, add=True)

        def body(j, carry):
            pltpu.async_copy(onesv, deg_sh.at[colv.at[j]], sem, add=True)
            wait_one()
            return carry

        lax.fori_loop(1, rpt, body, 0)
        wait_one()
        plsc.subcore_barrier()
        pltpu.sync_copy(deg_sh.at[pl.ds(s * SLICE, SLICE)],
                        out_hbm.at[c, pl.ds(s * SLICE, SLICE)])

    return deg_k(coli, ones128, zeros128)


def _agg_call(hs, rowi, coli, zeros128):
    chunks = rowi.shape[0]
    rpt = chunks // NW

    @functools.partial(
        pl.kernel,
        out_type=jax.ShapeDtypeStruct((NC, NP, FEAT), jnp.float32),
        mesh=_sc_mesh(),
        scratch_types=[
            pltpu.VMEM((16, 128), jnp.int32),
            pltpu.VMEM((16, 128), jnp.int32),
            pltpu.VMEM((128, FEAT), jnp.float32),
            pltpu.VMEM((128, FEAT), jnp.float32),
            pltpu.VMEM_SHARED((NP, FEAT), jnp.float32),
            pltpu.SemaphoreType.DMA,
        ],
    )
    def agg_k(hs_hbm, rowi_hbm, coli_hbm, zeros_hbm, out_hbm,
              rowv, colv, gbuf0, gbuf1, acc_sh, sem):
        c = lax.axis_index("c")
        s = lax.axis_index("s")
        wid = c * NT + s
        seg = 16
        nseg = rpt // seg
        pltpu.sync_copy(zeros_hbm, acc_sh.at[pl.ds(s * SLICE, SLICE)])
        plsc.subcore_barrier()

        def seg_body(g, carry):
            base = wid * rpt + g * seg
            pltpu.sync_copy(rowi_hbm.at[pl.ds(base, seg)], rowv)
            pltpu.sync_copy(coli_hbm.at[pl.ds(base, seg)], colv)
            pltpu.async_copy(hs_hbm.at[rowv.at[0]], gbuf0, sem)

            def body(t, c2):
                j0 = t * 2
                pltpu.make_async_copy(hs_hbm.at[rowv.at[0]], gbuf0, sem).wait()
                pltpu.async_copy(hs_hbm.at[rowv.at[j0 + 1]], gbuf1, sem)
                pltpu.sync_copy(gbuf0, acc_sh.at[colv.at[j0]], add=True)
                pltpu.make_async_copy(hs_hbm.at[rowv.at[0]], gbuf1, sem).wait()

                @pl.when(t + 1 < seg // 2)
                def _():
                    pltpu.async_copy(hs_hbm.at[rowv.at[j0 + 2]], gbuf0, sem)

                pltpu.sync_copy(gbuf1, acc_sh.at[colv.at[j0 + 1]], add=True)
                return c2

            lax.fori_loop(0, seg // 2, body, 0)
            return carry

        lax.fori_loop(0, nseg, seg_body, 0)
        plsc.subcore_barrier()
        pltpu.sync_copy(acc_sh.at[pl.ds(s * SLICE, SLICE)],
                        out_hbm.at[c, pl.ds(s * SLICE, SLICE)])

    return agg_k(hs, rowi, coli, zeros128)


def _first_tc(degp, x, W1):
    def body(degp_ref, x_ref, w_ref, hs_ref, disb_ref):
        deg = degp_ref[0, :, 0:1] + degp_ref[1, :, 0:1] + 1.0
        dis = lax.rsqrt(deg)
        h = jnp.dot(x_ref[...], w_ref[...], preferred_element_type=jnp.float32)
        hs_ref[...] = h * dis
        disb_ref[...] = jnp.broadcast_to(dis, (BLK, FEAT))

    return pl.pallas_call(
        body,
        grid=(GRID,),
        in_specs=[
            pl.BlockSpec((NC, BLK, FEAT), lambda i: (0, i, 0)),
            pl.BlockSpec((BLK, FEAT), lambda i: (i, 0)),
            pl.BlockSpec((FEAT, FEAT), lambda i: (0, 0)),
        ],
        out_specs=[
            pl.BlockSpec((BLK, FEAT), lambda i: (i, 0)),
            pl.BlockSpec((BLK, FEAT), lambda i: (i, 0)),
        ],
        out_shape=[
            jax.ShapeDtypeStruct((N_NODES, FEAT), jnp.float32),
            jax.ShapeDtypeStruct((N_NODES, FEAT), jnp.float32),
        ],
    )(degp, x, W1)


def _mid_tc(accp, hs, disb, b, W):
    def body(accp_ref, hs_ref, disb_ref, b_ref, w_ref, out_ref):
        agg = accp_ref[0] + accp_ref[1] + hs_ref[...]
        xn = jnp.maximum(disb_ref[...] * agg + b_ref[...], 0.0)
        h = jnp.dot(xn, w_ref[...], preferred_element_type=jnp.float32)
        out_ref[...] = h * disb_ref[...]

    return pl.pallas_call(
        body,
        grid=(GRID,),
        in_specs=[
            pl.BlockSpec((NC, BLK, FEAT), lambda i: (0, i, 0)),
            pl.BlockSpec((BLK, FEAT), lambda i: (i, 0)),
            pl.BlockSpec((BLK, FEAT), lambda i: (i, 0)),
            pl.BlockSpec((1, FEAT), lambda i: (0, 0)),
            pl.BlockSpec((FEAT, FEAT), lambda i: (0, 0)),
        ],
        out_specs=pl.BlockSpec((BLK, FEAT), lambda i: (i, 0)),
        out_shape=jax.ShapeDtypeStruct((N_NODES, FEAT), jnp.float32),
    )(accp, hs, disb, b, W)


def _final_tc(accp, hs, disb, b, batch3, Wf1, bf1, Wf2, bf2):
    h3 = Wf1.shape[1]
    nout = Wf2.shape[1]

    def body(accp_ref, hs_ref, disb_ref, b_ref, batch_ref,
             wf1_ref, bf1_ref, wf2_ref, bf2_ref, out_ref, pooled, counts):
        i = pl.program_id(0)

        @pl.when(i == 0)
        def _():
            pooled[...] = jnp.zeros_like(pooled)
            counts[...] = jnp.zeros_like(counts)

        agg = accp_ref[0] + accp_ref[1] + hs_ref[...]
        x3 = jnp.maximum(disb_ref[...] * agg + b_ref[...], 0.0)
        batch_row = batch_ref[0]
        giota = lax.broadcasted_iota(jnp.int32, (N_GRAPHS, BLK), 0)
        onehot_t = (giota == batch_row).astype(jnp.float32)
        pooled[...] += lax.dot_general(
            onehot_t, x3, (((1,), (0,)), ((), ())),
            preferred_element_type=jnp.float32)
        counts[...] += jnp.broadcast_to(
            jnp.sum(onehot_t, axis=1, keepdims=True), (N_GRAPHS, FEAT))

        @pl.when(i == GRID - 1)
        def _():
            mean = pooled[...] / jnp.maximum(counts[...], 1.0)
            hmid = jnp.dot(mean, wf1_ref[...],
                           preferred_element_type=jnp.float32) + bf1_ref[...]
            out_ref[...] = jnp.dot(hmid, wf2_ref[...],
                                   preferred_element_type=jnp.float32) + bf2_ref[...]

    return pl.pallas_call(
        body,
        grid=(GRID,),
        in_specs=[
            pl.BlockSpec((NC, BLK, FEAT), lambda i: (0, i, 0)),
            pl.BlockSpec((BLK, FEAT), lambda i: (i, 0)),
            pl.BlockSpec((BLK, FEAT), lambda i: (i, 0)),
            pl.BlockSpec((1, FEAT), lambda i: (0, 0)),
            pl.BlockSpec((1, 1, BLK), lambda i: (i, 0, 0)),
            pl.BlockSpec((FEAT, h3), lambda i: (0, 0)),
            pl.BlockSpec((1, h3), lambda i: (0, 0)),
            pl.BlockSpec((h3, nout), lambda i: (0, 0)),
            pl.BlockSpec((1, nout), lambda i: (0, 0)),
        ],
        out_specs=pl.BlockSpec((N_GRAPHS, nout), lambda i: (0, 0)),
        out_shape=jax.ShapeDtypeStruct((N_GRAPHS, nout), jnp.float32),
        scratch_shapes=[
            pltpu.VMEM((N_GRAPHS, FEAT), jnp.float32),
            pltpu.VMEM((N_GRAPHS, FEAT), jnp.float32),
        ],
    )(accp, hs, disb, b, batch3, Wf1, bf1, Wf2, bf2)


def kernel(x, edge_index, batch, W1, b1, W2, b2, Wf1, bf1, Wf2, bf2):
    row = edge_index[0]
    col = edge_index[1]
    n_edges = row.shape[0]
    quantum = NW * 128 * 8
    ep = ((n_edges + quantum - 1) // quantum) * quantum
    pad = ep - n_edges
    pad_iota = jnp.arange(pad, dtype=jnp.int32)
    rowp = jnp.concatenate(
        [row, pad_iota % N_NODES]).reshape(ep // 128, 128)
    colp = jnp.concatenate(
        [col, N_NODES + pad_iota % (NP - N_NODES)]).reshape(ep // 128, 128)

    ones128 = jnp.ones((128, FEAT), jnp.float32)
    zeros128 = jnp.zeros((SLICE, FEAT), jnp.float32)
    b1r = b1.reshape(1, FEAT)
    b2r = b2.reshape(1, FEAT)
    bf1r = bf1.reshape(1, -1)
    bf2r = bf2.reshape(1, -1)
    batch3 = batch.reshape(GRID, 1, BLK)

    degp = _deg_call(colp, ones128, zeros128)
    hs1, disb = _first_tc(degp, x, W1)
    a1 = _agg_call(hs1, rowp, colp, zeros128)
    hs2 = _mid_tc(a1, hs1, disb, b1r, W2)
    a2 = _agg_call(hs2, rowp, colp, zeros128)
    return _final_tc(a2, hs2, disb, b2r, batch3, Wf1, bf1r, Wf2, bf2r)


# speedup vs baseline: 1.2176x; 1.1363x over previous
import functools

import jax
import jax.numpy as jnp
from jax import lax
from jax.experimental import pallas as pl
from jax.experimental.pallas import tpu as pltpu
from jax.experimental.pallas import tpu_sc as plsc

N_NODES = 10000
FEAT = 128
N_GRAPHS = 64

NC = 2
NT = 16
NW = NC * NT
NP = 10112
SLICE = NP // NT

BLK = 1000
GRID = N_NODES // BLK


def _sc_mesh():
    return plsc.VectorSubcoreMesh(core_axis_name="c", subcore_axis_name="s")


def _deg_call(coli, zeros1d):
    chunks = coli.shape[0]
    rpt = chunks // NW
    npd = 10240
    tile_rows = npd // NT

    @functools.partial(
        pl.kernel,
        out_type=jax.ShapeDtypeStruct((NC * npd,), jnp.float32),
        mesh=_sc_mesh(),
        scratch_types=[
            pltpu.VMEM((rpt, 128), jnp.int32),
            pltpu.VMEM((128,), jnp.float32),
            pltpu.VMEM_SHARED((npd,), jnp.float32),
            pltpu.SemaphoreType.DMA,
        ],
    )
    def deg_k(coli_hbm, zeros_hbm, out_hbm, colv, onesv, deg_sh, sem):
        c = lax.axis_index("c")
        s = lax.axis_index("s")
        wid = c * NT + s
        pltpu.sync_copy(zeros_hbm.at[pl.ds(s * tile_rows, tile_rows)],
                        deg_sh.at[pl.ds(s * tile_rows, tile_rows)])
        pltpu.sync_copy(coli_hbm.at[pl.ds(wid * rpt, rpt)], colv)
        for k in range(8):
            onesv[pl.ds(k * 16, 16)] = jnp.ones((16,), jnp.float32)
        plsc.subcore_barrier()

        def wait_one():
            pltpu.make_async_copy(zeros_hbm.at[pl.ds(0, 128)], onesv,
                                  sem).wait()

        pltpu.async_copy(onesv, deg_sh.at[colv.at[0]], sem, add=True)

        def body(j, carry):
            pltpu.async_copy(onesv, deg_sh.at[colv.at[j]], sem, add=True)
            wait_one()
            return carry

        lax.fori_loop(1, rpt, body, 0)
        wait_one()
        plsc.subcore_barrier()
        pltpu.sync_copy(deg_sh.at[pl.ds(s * tile_rows, tile_rows)],
                        out_hbm.at[pl.ds(c * npd + s * tile_rows, tile_rows)])

    return deg_k(coli, zeros1d)


def _agg_call(hs, rowi, coli, zeros128):
    chunks = rowi.shape[0]
    rpt = chunks // NW

    @functools.partial(
        pl.kernel,
        out_type=jax.ShapeDtypeStruct((NC, NP, FEAT), jnp.float32),
        mesh=_sc_mesh(),
        scratch_types=[
            pltpu.VMEM((16, 128), jnp.int32),
            pltpu.VMEM((16, 128), jnp.int32),
            pltpu.VMEM((128, FEAT), jnp.float32),
            pltpu.VMEM((128, FEAT), jnp.float32),
            pltpu.VMEM_SHARED((NP, FEAT), jnp.float32),
            pltpu.SemaphoreType.DMA,
        ],
    )
    def agg_k(hs_hbm, rowi_hbm, coli_hbm, zeros_hbm, out_hbm,
              rowv, colv, gbuf0, gbuf1, acc_sh, sem):
        c = lax.axis_index("c")
        s = lax.axis_index("s")
        wid = c * NT + s
        seg = 16
        nseg = rpt // seg
        pltpu.sync_copy(zeros_hbm, acc_sh.at[pl.ds(s * SLICE, SLICE)])
        plsc.subcore_barrier()

        def seg_body(g, carry):
            base = wid * rpt + g * seg
            pltpu.sync_copy(rowi_hbm.at[pl.ds(base, seg)], rowv)
            pltpu.sync_copy(coli_hbm.at[pl.ds(base, seg)], colv)
            pltpu.async_copy(hs_hbm.at[rowv.at[0]], gbuf0, sem)

            def body(t, c2):
                j0 = t * 2
                pltpu.make_async_copy(hs_hbm.at[rowv.at[0]], gbuf0, sem).wait()
                pltpu.async_copy(hs_hbm.at[rowv.at[j0 + 1]], gbuf1, sem)
                pltpu.sync_copy(gbuf0, acc_sh.at[colv.at[j0]], add=True)
                pltpu.make_async_copy(hs_hbm.at[rowv.at[0]], gbuf1, sem).wait()

                @pl.when(t + 1 < seg // 2)
                def _():
                    pltpu.async_copy(hs_hbm.at[rowv.at[j0 + 2]], gbuf0, sem)

                pltpu.sync_copy(gbuf1, acc_sh.at[colv.at[j0 + 1]], add=True)
                return c2

            lax.fori_loop(0, seg // 2, body, 0)
            return carry

        lax.fori_loop(0, nseg, seg_body, 0)
        plsc.subcore_barrier()
        pltpu.sync_copy(acc_sh.at[pl.ds(s * SLICE, SLICE)],
                        out_hbm.at[c, pl.ds(s * SLICE, SLICE)])

    return agg_k(hs, rowi, coli, zeros128)


def _first_tc(degb0, degb1, x, W1):
    def body(d0_ref, d1_ref, x_ref, w_ref, hs_ref, disb_ref):
        deg = d0_ref[:, 0:1] + d1_ref[:, 0:1] + 1.0
        dis = lax.rsqrt(deg)
        h = jnp.dot(x_ref[...], w_ref[...], preferred_element_type=jnp.float32)
        hs_ref[...] = h * dis
        disb_ref[...] = jnp.broadcast_to(dis, (BLK, FEAT))

    return pl.pallas_call(
        body,
        grid=(GRID,),
        in_specs=[
            pl.BlockSpec((BLK, FEAT), lambda i: (i, 0)),
            pl.BlockSpec((BLK, FEAT), lambda i: (i, 0)),
            pl.BlockSpec((BLK, FEAT), lambda i: (i, 0)),
            pl.BlockSpec((FEAT, FEAT), lambda i: (0, 0)),
        ],
        out_specs=[
            pl.BlockSpec((BLK, FEAT), lambda i: (i, 0)),
            pl.BlockSpec((BLK, FEAT), lambda i: (i, 0)),
        ],
        out_shape=[
            jax.ShapeDtypeStruct((N_NODES, FEAT), jnp.float32),
            jax.ShapeDtypeStruct((N_NODES, FEAT), jnp.float32),
        ],
    )(degb0, degb1, x, W1)


def _mid_tc(accp, hs, disb, b, W):
    def body(accp_ref, hs_ref, disb_ref, b_ref, w_ref, out_ref):
        agg = accp_ref[0] + accp_ref[1] + hs_ref[...]
        xn = jnp.maximum(disb_ref[...] * agg + b_ref[...], 0.0)
        h = jnp.dot(xn, w_ref[...], preferred_element_type=jnp.float32)
        out_ref[...] = h * disb_ref[...]

    return pl.pallas_call(
        body,
        grid=(GRID,),
        in_specs=[
            pl.BlockSpec((NC, BLK, FEAT), lambda i: (0, i, 0)),
            pl.BlockSpec((BLK, FEAT), lambda i: (i, 0)),
            pl.BlockSpec((BLK, FEAT), lambda i: (i, 0)),
            pl.BlockSpec((1, FEAT), lambda i: (0, 0)),
            pl.BlockSpec((FEAT, FEAT), lambda i: (0, 0)),
        ],
        out_specs=pl.BlockSpec((BLK, FEAT), lambda i: (i, 0)),
        out_shape=jax.ShapeDtypeStruct((N_NODES, FEAT), jnp.float32),
    )(accp, hs, disb, b, W)


def _final_tc(accp, hs, disb, b, batch3, Wf1, bf1, Wf2, bf2):
    h3 = Wf1.shape[1]
    nout = Wf2.shape[1]

    def body(accp_ref, hs_ref, disb_ref, b_ref, batch_ref,
             wf1_ref, bf1_ref, wf2_ref, bf2_ref, out_ref, pooled, counts):
        i = pl.program_id(0)

        @pl.when(i == 0)
        def _():
            pooled[...] = jnp.zeros_like(pooled)
            counts[...] = jnp.zeros_like(counts)

        agg = accp_ref[0] + accp_ref[1] + hs_ref[...]
        x3 = jnp.maximum(disb_ref[...] * agg + b_ref[...], 0.0)
        batch_row = batch_ref[0]
        giota = lax.broadcasted_iota(jnp.int32, (N_GRAPHS, BLK), 0)
        onehot_t = (giota == batch_row).astype(jnp.float32)
        pooled[...] += lax.dot_general(
            onehot_t, x3, (((1,), (0,)), ((), ())),
            preferred_element_type=jnp.float32)
        counts[...] += jnp.broadcast_to(
            jnp.sum(onehot_t, axis=1, keepdims=True), (N_GRAPHS, FEAT))

        @pl.when(i == GRID - 1)
        def _():
            mean = pooled[...] / jnp.maximum(counts[...], 1.0)
            hmid = jnp.dot(mean, wf1_ref[...],
                           preferred_element_type=jnp.float32) + bf1_ref[...]
            out_ref[...] = jnp.dot(hmid, wf2_ref[...],
                                   preferred_element_type=jnp.float32) + bf2_ref[...]

    return pl.pallas_call(
        body,
        grid=(GRID,),
        in_specs=[
            pl.BlockSpec((NC, BLK, FEAT), lambda i: (0, i, 0)),
            pl.BlockSpec((BLK, FEAT), lambda i: (i, 0)),
            pl.BlockSpec((BLK, FEAT), lambda i: (i, 0)),
            pl.BlockSpec((1, FEAT), lambda i: (0, 0)),
            pl.BlockSpec((1, 1, BLK), lambda i: (i, 0, 0)),
            pl.BlockSpec((FEAT, h3), lambda i: (0, 0)),
            pl.BlockSpec((1, h3), lambda i: (0, 0)),
            pl.BlockSpec((h3, nout), lambda i: (0, 0)),
            pl.BlockSpec((1, nout), lambda i: (0, 0)),
        ],
        out_specs=pl.BlockSpec((N_GRAPHS, nout), lambda i: (0, 0)),
        out_shape=jax.ShapeDtypeStruct((N_GRAPHS, nout), jnp.float32),
        scratch_shapes=[
            pltpu.VMEM((N_GRAPHS, FEAT), jnp.float32),
            pltpu.VMEM((N_GRAPHS, FEAT), jnp.float32),
        ],
    )(accp, hs, disb, b, batch3, Wf1, bf1, Wf2, bf2)


def kernel(x, edge_index, batch, W1, b1, W2, b2, Wf1, bf1, Wf2, bf2):
    row = edge_index[0]
    col = edge_index[1]
    n_edges = row.shape[0]
    quantum = NW * 128 * 8
    ep = ((n_edges + quantum - 1) // quantum) * quantum
    pad = ep - n_edges
    pad_iota = jnp.arange(pad, dtype=jnp.int32)
    rowp = jnp.concatenate(
        [row, pad_iota % N_NODES]).reshape(ep // 128, 128)
    colp = jnp.concatenate(
        [col, N_NODES + pad_iota % (NP - N_NODES)]).reshape(ep // 128, 128)

    zeros1d = jnp.zeros((10240,), jnp.float32)
    zeros128 = jnp.zeros((SLICE, FEAT), jnp.float32)
    b1r = b1.reshape(1, FEAT)
    b2r = b2.reshape(1, FEAT)
    bf1r = bf1.reshape(1, -1)
    bf2r = bf2.reshape(1, -1)
    batch3 = batch.reshape(GRID, 1, BLK)

    degp = _deg_call(colp, zeros1d)
    dp = degp.reshape(NC, 10240)[:, :N_NODES]
    degb0 = jnp.broadcast_to(dp[0][:, None], (N_NODES, FEAT))
    degb1 = jnp.broadcast_to(dp[1][:, None], (N_NODES, FEAT))
    hs1, disb = _first_tc(degb0, degb1, x, W1)
    a1 = _agg_call(hs1, rowp, colp, zeros128)
    hs2 = _mid_tc(a1, hs1, disb, b1r, W2)
    a2 = _agg_call(hs2, rowp, colp, zeros128)
    return _final_tc(a2, hs2, disb, b2r, batch3, Wf1, bf1r, Wf2, bf2r)


# speedup vs baseline: 1.2513x; 1.0277x over previous
import functools

import jax
import jax.numpy as jnp
from jax import lax
from jax.experimental import pallas as pl
from jax.experimental.pallas import tpu as pltpu
from jax.experimental.pallas import tpu_sc as plsc

N_NODES = 10000
FEAT = 128
N_GRAPHS = 64

NC = 2
NT = 16
NW = NC * NT
NP = 10112
SLICE = NP // NT

BLK = 1000
GRID = N_NODES // BLK


def _sc_mesh():
    return plsc.VectorSubcoreMesh(core_axis_name="c", subcore_axis_name="s")


def _deg_call(coli, zeros1d):
    chunks = coli.shape[0]
    rpt = chunks // NW
    npd = 10240
    tile_rows = npd // NT

    @functools.partial(
        pl.kernel,
        out_type=jax.ShapeDtypeStruct((NC * npd,), jnp.float32),
        mesh=_sc_mesh(),
        scratch_types=[
            pltpu.VMEM((rpt, 128), jnp.int32),
            pltpu.VMEM((128,), jnp.float32),
            pltpu.VMEM_SHARED((npd,), jnp.float32),
            pltpu.SemaphoreType.DMA,
        ],
    )
    def deg_k(coli_hbm, zeros_hbm, out_hbm, colv, onesv, deg_sh, sem):
        c = lax.axis_index("c")
        s = lax.axis_index("s")
        wid = c * NT + s
        pltpu.sync_copy(zeros_hbm.at[pl.ds(s * tile_rows, tile_rows)],
                        deg_sh.at[pl.ds(s * tile_rows, tile_rows)])
        pltpu.sync_copy(coli_hbm.at[pl.ds(wid * rpt, rpt)], colv)
        for k in range(8):
            onesv[pl.ds(k * 16, 16)] = jnp.ones((16,), jnp.float32)
        plsc.subcore_barrier()

        def wait_one():
            pltpu.make_async_copy(zeros_hbm.at[pl.ds(0, 128)], onesv,
                                  sem).wait()

        pltpu.async_copy(onesv, deg_sh.at[colv.at[0]], sem, add=True)

        def body(j, carry):
            pltpu.async_copy(onesv, deg_sh.at[colv.at[j]], sem, add=True)
            wait_one()
            return carry

        lax.fori_loop(1, rpt, body, 0)
        wait_one()
        plsc.subcore_barrier()
        pltpu.sync_copy(deg_sh.at[pl.ds(s * tile_rows, tile_rows)],
                        out_hbm.at[pl.ds(c * npd + s * tile_rows, tile_rows)])

    return deg_k(coli, zeros1d)


def _agg_call(hs, rowi, coli, zeros128):
    chunks = rowi.shape[0]
    rpt = chunks // NW

    @functools.partial(
        pl.kernel,
        out_type=jax.ShapeDtypeStruct((NC, NP, FEAT), jnp.float32),
        mesh=_sc_mesh(),
        scratch_types=[
            pltpu.VMEM((16, 128), jnp.int32),
            pltpu.VMEM((16, 128), jnp.int32),
            pltpu.VMEM((128, FEAT), jnp.float32),
            pltpu.VMEM((128, FEAT), jnp.float32),
            pltpu.VMEM_SHARED((NP, FEAT), jnp.float32),
            pltpu.SemaphoreType.DMA,
        ],
    )
    def agg_k(hs_hbm, rowi_hbm, coli_hbm, zeros_hbm, out_hbm,
              rowv, colv, gbuf0, gbuf1, acc_sh, sem):
        c = lax.axis_index("c")
        s = lax.axis_index("s")
        wid = c * NT + s
        seg = 16
        nseg = rpt // seg
        pltpu.sync_copy(zeros_hbm, acc_sh.at[pl.ds(s * SLICE, SLICE)])
        plsc.subcore_barrier()

        def seg_body(g, carry):
            base = wid * rpt + g * seg
            pltpu.sync_copy(rowi_hbm.at[pl.ds(base, seg)], rowv)
            pltpu.sync_copy(coli_hbm.at[pl.ds(base, seg)], colv)
            pltpu.async_copy(hs_hbm.at[rowv.at[0]], gbuf0, sem)

            def body(t, c2):
                j0 = t * 2
                pltpu.make_async_copy(hs_hbm.at[rowv.at[0]], gbuf0, sem).wait()
                pltpu.async_copy(hs_hbm.at[rowv.at[j0 + 1]], gbuf1, sem)
                pltpu.sync_copy(gbuf0, acc_sh.at[colv.at[j0]], add=True)
                pltpu.make_async_copy(hs_hbm.at[rowv.at[0]], gbuf1, sem).wait()

                @pl.when(t + 1 < seg // 2)
                def _():
                    pltpu.async_copy(hs_hbm.at[rowv.at[j0 + 2]], gbuf0, sem)

                pltpu.sync_copy(gbuf1, acc_sh.at[colv.at[j0 + 1]], add=True)
                return c2

            lax.fori_loop(0, seg // 2, body, 0)
            return carry

        lax.fori_loop(0, nseg, seg_body, 0)
        plsc.subcore_barrier()
        pltpu.sync_copy(acc_sh.at[pl.ds(s * SLICE, SLICE)],
                        out_hbm.at[c, pl.ds(s * SLICE, SLICE)])

    return agg_k(hs, rowi, coli, zeros128)


def _first_tc(dp3, x, W1):
    def body(dp_ref, x_ref, w_ref, hs_ref, disb_ref):
        dvals = dp_ref[0]
        deg = dvals[0:1, :] + dvals[1:2, :] + 1.0
        dis = jnp.transpose(lax.rsqrt(deg), (1, 0))
        h = jnp.dot(x_ref[...], w_ref[...], preferred_element_type=jnp.float32)
        hs_ref[...] = h * dis
        disb_ref[...] = jnp.broadcast_to(dis, (BLK, FEAT))

    return pl.pallas_call(
        body,
        grid=(GRID,),
        in_specs=[
            pl.BlockSpec((1, NC, BLK), lambda i: (i, 0, 0)),
            pl.BlockSpec((BLK, FEAT), lambda i: (i, 0)),
            pl.BlockSpec((FEAT, FEAT), lambda i: (0, 0)),
        ],
        out_specs=[
            pl.BlockSpec((BLK, FEAT), lambda i: (i, 0)),
            pl.BlockSpec((BLK, FEAT), lambda i: (i, 0)),
        ],
        out_shape=[
            jax.ShapeDtypeStruct((N_NODES, FEAT), jnp.float32),
            jax.ShapeDtypeStruct((N_NODES, FEAT), jnp.float32),
        ],
    )(dp3, x, W1)


def _mid_tc(accp, hs, disb, b, W):
    def body(accp_ref, hs_ref, disb_ref, b_ref, w_ref, out_ref):
        agg = accp_ref[0] + accp_ref[1] + hs_ref[...]
        xn = jnp.maximum(disb_ref[...] * agg + b_ref[...], 0.0)
        h = jnp.dot(xn, w_ref[...], preferred_element_type=jnp.float32)
        out_ref[...] = h * disb_ref[...]

    return pl.pallas_call(
        body,
        grid=(GRID,),
        in_specs=[
            pl.BlockSpec((NC, BLK, FEAT), lambda i: (0, i, 0)),
            pl.BlockSpec((BLK, FEAT), lambda i: (i, 0)),
            pl.BlockSpec((BLK, FEAT), lambda i: (i, 0)),
            pl.BlockSpec((1, FEAT), lambda i: (0, 0)),
            pl.BlockSpec((FEAT, FEAT), lambda i: (0, 0)),
        ],
        out_specs=pl.BlockSpec((BLK, FEAT), lambda i: (i, 0)),
        out_shape=jax.ShapeDtypeStruct((N_NODES, FEAT), jnp.float32),
    )(accp, hs, disb, b, W)


def _final_tc(accp, hs, disb, b, batch3, Wf1, bf1, Wf2, bf2):
    h3 = Wf1.shape[1]
    nout = Wf2.shape[1]

    def body(accp_ref, hs_ref, disb_ref, b_ref, batch_ref,
             wf1_ref, bf1_ref, wf2_ref, bf2_ref, out_ref, pooled, counts):
        i = pl.program_id(0)

        @pl.when(i == 0)
        def _():
            pooled[...] = jnp.zeros_like(pooled)
            counts[...] = jnp.zeros_like(counts)

        agg = accp_ref[0] + accp_ref[1] + hs_ref[...]
        x3 = jnp.maximum(disb_ref[...] * agg + b_ref[...], 0.0)
        batch_row = batch_ref[0]
        giota = lax.broadcasted_iota(jnp.int32, (N_GRAPHS, BLK), 0)
        onehot_t = (giota == batch_row).astype(jnp.float32)
        pooled[...] += lax.dot_general(
            onehot_t, x3, (((1,), (0,)), ((), ())),
            preferred_element_type=jnp.float32)
        counts[...] += jnp.broadcast_to(
            jnp.sum(onehot_t, axis=1, keepdims=True), (N_GRAPHS, FEAT))

        @pl.when(i == GRID - 1)
        def _():
            mean = pooled[...] / jnp.maximum(counts[...], 1.0)
            hmid = jnp.dot(mean, wf1_ref[...],
                           preferred_element_type=jnp.float32) + bf1_ref[...]
            out_ref[...] = jnp.dot(hmid, wf2_ref[...],
                                   preferred_element_type=jnp.float32) + bf2_ref[...]

    return pl.pallas_call(
        body,
        grid=(GRID,),
        in_specs=[
            pl.BlockSpec((NC, BLK, FEAT), lambda i: (0, i, 0)),
            pl.BlockSpec((BLK, FEAT), lambda i: (i, 0)),
            pl.BlockSpec((BLK, FEAT), lambda i: (i, 0)),
            pl.BlockSpec((1, FEAT), lambda i: (0, 0)),
            pl.BlockSpec((1, 1, BLK), lambda i: (i, 0, 0)),
            pl.BlockSpec((FEAT, h3), lambda i: (0, 0)),
            pl.BlockSpec((1, h3), lambda i: (0, 0)),
            pl.BlockSpec((h3, nout), lambda i: (0, 0)),
            pl.BlockSpec((1, nout), lambda i: (0, 0)),
        ],
        out_specs=pl.BlockSpec((N_GRAPHS, nout), lambda i: (0, 0)),
        out_shape=jax.ShapeDtypeStruct((N_GRAPHS, nout), jnp.float32),
        scratch_shapes=[
            pltpu.VMEM((N_GRAPHS, FEAT), jnp.float32),
            pltpu.VMEM((N_GRAPHS, FEAT), jnp.float32),
        ],
    )(accp, hs, disb, b, batch3, Wf1, bf1, Wf2, bf2)


def kernel(x, edge_index, batch, W1, b1, W2, b2, Wf1, bf1, Wf2, bf2):
    row = edge_index[0]
    col = edge_index[1]
    n_edges = row.shape[0]
    quantum = NW * 128 * 8
    ep = ((n_edges + quantum - 1) // quantum) * quantum
    pad = ep - n_edges
    pad_iota = jnp.arange(pad, dtype=jnp.int32)
    rowp = jnp.concatenate(
        [row, pad_iota % N_NODES]).reshape(ep // 128, 128)
    colp = jnp.concatenate(
        [col, N_NODES + pad_iota % (NP - N_NODES)]).reshape(ep // 128, 128)

    zeros1d = jnp.zeros((10240,), jnp.float32)
    zeros128 = jnp.zeros((SLICE, FEAT), jnp.float32)
    b1r = b1.reshape(1, FEAT)
    b2r = b2.reshape(1, FEAT)
    bf1r = bf1.reshape(1, -1)
    bf2r = bf2.reshape(1, -1)
    batch3 = batch.reshape(GRID, 1, BLK)

    degp = _deg_call(colp, zeros1d)
    dp3 = (degp.reshape(NC, 10240)[:, :N_NODES]
           .reshape(NC, GRID, BLK).transpose(1, 0, 2))
    hs1, disb = _first_tc(dp3, x, W1)
    a1 = _agg_call(hs1, rowp, colp, zeros128)
    hs2 = _mid_tc(a1, hs1, disb, b1r, W2)
    a2 = _agg_call(hs2, rowp, colp, zeros128)
    return _final_tc(a2, hs2, disb, b2r, batch3, Wf1, bf1r, Wf2, bf2r)


# speedup vs baseline: 1.2540x; 1.0021x over previous
import functools

import jax
import jax.numpy as jnp
from jax import lax
from jax.experimental import pallas as pl
from jax.experimental.pallas import tpu as pltpu
from jax.experimental.pallas import tpu_sc as plsc

N_NODES = 10000
FEAT = 128
N_GRAPHS = 64

NC = 2
NT = 16
NW = NC * NT
NP = 10112
SLICE = NP // NT

BLK = 1000
GRID = N_NODES // BLK


def _sc_mesh():
    return plsc.VectorSubcoreMesh(core_axis_name="c", subcore_axis_name="s")


def _deg_call(coli, zeros1d):
    chunks = coli.shape[0]
    rpt = chunks // NW
    npd = 10240
    tile_rows = npd // NT

    @functools.partial(
        pl.kernel,
        out_type=jax.ShapeDtypeStruct((NC * npd,), jnp.float32),
        mesh=_sc_mesh(),
        scratch_types=[
            pltpu.VMEM((rpt, 128), jnp.int32),
            pltpu.VMEM((128,), jnp.float32),
            pltpu.VMEM_SHARED((npd,), jnp.float32),
            pltpu.SemaphoreType.DMA,
        ],
    )
    def deg_k(coli_hbm, zeros_hbm, out_hbm, colv, onesv, deg_sh, sem):
        c = lax.axis_index("c")
        s = lax.axis_index("s")
        wid = c * NT + s
        pltpu.sync_copy(zeros_hbm.at[pl.ds(s * tile_rows, tile_rows)],
                        deg_sh.at[pl.ds(s * tile_rows, tile_rows)])
        pltpu.sync_copy(coli_hbm.at[pl.ds(wid * rpt, rpt)], colv)
        for k in range(8):
            onesv[pl.ds(k * 16, 16)] = jnp.ones((16,), jnp.float32)
        plsc.subcore_barrier()

        def wait_one():
            pltpu.make_async_copy(zeros_hbm.at[pl.ds(0, 128)], onesv,
                                  sem).wait()

        pltpu.async_copy(onesv, deg_sh.at[colv.at[0]], sem, add=True)

        def body(j, carry):
            pltpu.async_copy(onesv, deg_sh.at[colv.at[j]], sem, add=True)
            wait_one()
            return carry

        lax.fori_loop(1, rpt, body, 0)
        wait_one()
        plsc.subcore_barrier()
        pltpu.sync_copy(deg_sh.at[pl.ds(s * tile_rows, tile_rows)],
                        out_hbm.at[pl.ds(c * npd + s * tile_rows, tile_rows)])

    return deg_k(coli, zeros1d)


def _agg_call(hs, rowi, coli, zeros128):
    chunks = rowi.shape[0]
    rpt = chunks // NW

    @functools.partial(
        pl.kernel,
        out_type=jax.ShapeDtypeStruct((NC, NP, FEAT), jnp.float32),
        mesh=_sc_mesh(),
        scratch_types=[
            pltpu.VMEM((16, 128), jnp.int32),
            pltpu.VMEM((16, 128), jnp.int32),
            pltpu.VMEM((128, FEAT), jnp.float32),
            pltpu.VMEM((128, FEAT), jnp.float32),
            pltpu.VMEM_SHARED((NP, FEAT), jnp.float32),
            pltpu.SemaphoreType.DMA,
        ],
    )
    def agg_k(hs_hbm, rowi_hbm, coli_hbm, zeros_hbm, out_hbm,
              rowv, colv, gbuf0, gbuf1, acc_sh, sem):
        c = lax.axis_index("c")
        s = lax.axis_index("s")
        wid = c * NT + s
        seg = 16
        nseg = rpt // seg
        pltpu.sync_copy(zeros_hbm, acc_sh.at[pl.ds(s * SLICE, SLICE)])
        plsc.subcore_barrier()

        def seg_body(g, carry):
            base = wid * rpt + g * seg
            pltpu.sync_copy(rowi_hbm.at[pl.ds(base, seg)], rowv)
            pltpu.sync_copy(coli_hbm.at[pl.ds(base, seg)], colv)
            pltpu.async_copy(hs_hbm.at[rowv.at[0]], gbuf0, sem)

            def body(t, c2):
                j0 = t * 2
                pltpu.make_async_copy(hs_hbm.at[rowv.at[0]], gbuf0, sem).wait()
                pltpu.async_copy(hs_hbm.at[rowv.at[j0 + 1]], gbuf1, sem)
                pltpu.sync_copy(gbuf0, acc_sh.at[colv.at[j0]], add=True)
                pltpu.make_async_copy(hs_hbm.at[rowv.at[0]], gbuf1, sem).wait()

                @pl.when(t + 1 < seg // 2)
                def _():
                    pltpu.async_copy(hs_hbm.at[rowv.at[j0 + 2]], gbuf0, sem)

                pltpu.sync_copy(gbuf1, acc_sh.at[colv.at[j0 + 1]], add=True)
                return c2

            lax.fori_loop(0, seg // 2, body, 0)
            return carry

        lax.fori_loop(0, nseg, seg_body, 0)
        plsc.subcore_barrier()
        pltpu.sync_copy(acc_sh.at[pl.ds(s * SLICE, SLICE)],
                        out_hbm.at[c, pl.ds(s * SLICE, SLICE)])

    return agg_k(hs, rowi, coli, zeros128)


def _dis_col(dp_ref):
    dvals = dp_ref[0]
    deg = dvals[0:1, :] + dvals[1:2, :] + 1.0
    return jnp.transpose(lax.rsqrt(deg), (1, 0))


def _first_tc(dp3, x, W1):
    def body(dp_ref, x_ref, w_ref, hs_ref):
        dis = _dis_col(dp_ref)
        h = jnp.dot(x_ref[...], w_ref[...], preferred_element_type=jnp.float32)
        hs_ref[...] = h * dis

    return pl.pallas_call(
        body,
        grid=(GRID,),
        in_specs=[
            pl.BlockSpec((1, NC, BLK), lambda i: (i, 0, 0)),
            pl.BlockSpec((BLK, FEAT), lambda i: (i, 0)),
            pl.BlockSpec((FEAT, FEAT), lambda i: (0, 0)),
        ],
        out_specs=pl.BlockSpec((BLK, FEAT), lambda i: (i, 0)),
        out_shape=jax.ShapeDtypeStruct((N_NODES, FEAT), jnp.float32),
    )(dp3, x, W1)


def _mid_tc(accp, hs, dp3, b, W):
    def body(accp_ref, hs_ref, dp_ref, b_ref, w_ref, out_ref):
        dis = _dis_col(dp_ref)
        agg = accp_ref[0] + accp_ref[1] + hs_ref[...]
        xn = jnp.maximum(dis * agg + b_ref[...], 0.0)
        h = jnp.dot(xn, w_ref[...], preferred_element_type=jnp.float32)
        out_ref[...] = h * dis

    return pl.pallas_call(
        body,
        grid=(GRID,),
        in_specs=[
            pl.BlockSpec((NC, BLK, FEAT), lambda i: (0, i, 0)),
            pl.BlockSpec((BLK, FEAT), lambda i: (i, 0)),
            pl.BlockSpec((1, NC, BLK), lambda i: (i, 0, 0)),
            pl.BlockSpec((1, FEAT), lambda i: (0, 0)),
            pl.BlockSpec((FEAT, FEAT), lambda i: (0, 0)),
        ],
        out_specs=pl.BlockSpec((BLK, FEAT), lambda i: (i, 0)),
        out_shape=jax.ShapeDtypeStruct((N_NODES, FEAT), jnp.float32),
    )(accp, hs, dp3, b, W)


def _final_tc(accp, hs, dp3, b, batch3, Wf1, bf1, Wf2, bf2):
    h3 = Wf1.shape[1]
    nout = Wf2.shape[1]

    def body(accp_ref, hs_ref, dp_ref, b_ref, batch_ref,
             wf1_ref, bf1_ref, wf2_ref, bf2_ref, out_ref, pooled, counts):
        i = pl.program_id(0)

        @pl.when(i == 0)
        def _():
            pooled[...] = jnp.zeros_like(pooled)
            counts[...] = jnp.zeros_like(counts)

        dis = _dis_col(dp_ref)
        agg = accp_ref[0] + accp_ref[1] + hs_ref[...]
        x3 = jnp.maximum(dis * agg + b_ref[...], 0.0)
        batch_row = batch_ref[0]
        giota = lax.broadcasted_iota(jnp.int32, (N_GRAPHS, BLK), 0)
        onehot_t = (giota == batch_row).astype(jnp.float32)
        pooled[...] += lax.dot_general(
            onehot_t, x3, (((1,), (0,)), ((), ())),
            preferred_element_type=jnp.float32)
        counts[...] += jnp.broadcast_to(
            jnp.sum(onehot_t, axis=1, keepdims=True), (N_GRAPHS, FEAT))

        @pl.when(i == GRID - 1)
        def _():
            mean = pooled[...] / jnp.maximum(counts[...], 1.0)
            hmid = jnp.dot(mean, wf1_ref[...],
                           preferred_element_type=jnp.float32) + bf1_ref[...]
            out_ref[...] = jnp.dot(hmid, wf2_ref[...],
                                   preferred_element_type=jnp.float32) + bf2_ref[...]

    return pl.pallas_call(
        body,
        grid=(GRID,),
        in_specs=[
            pl.BlockSpec((NC, BLK, FEAT), lambda i: (0, i, 0)),
            pl.BlockSpec((BLK, FEAT), lambda i: (i, 0)),
            pl.BlockSpec((1, NC, BLK), lambda i: (i, 0, 0)),
            pl.BlockSpec((1, FEAT), lambda i: (0, 0)),
            pl.BlockSpec((1, 1, BLK), lambda i: (i, 0, 0)),
            pl.BlockSpec((FEAT, h3), lambda i: (0, 0)),
            pl.BlockSpec((1, h3), lambda i: (0, 0)),
            pl.BlockSpec((h3, nout), lambda i: (0, 0)),
            pl.BlockSpec((1, nout), lambda i: (0, 0)),
        ],
        out_specs=pl.BlockSpec((N_GRAPHS, nout), lambda i: (0, 0)),
        out_shape=jax.ShapeDtypeStruct((N_GRAPHS, nout), jnp.float32),
        scratch_shapes=[
            pltpu.VMEM((N_GRAPHS, FEAT), jnp.float32),
            pltpu.VMEM((N_GRAPHS, FEAT), jnp.float32),
        ],
    )(accp, hs, dp3, b, batch3, Wf1, bf1, Wf2, bf2)


def kernel(x, edge_index, batch, W1, b1, W2, b2, Wf1, bf1, Wf2, bf2):
    row = edge_index[0]
    col = edge_index[1]
    n_edges = row.shape[0]
    quantum = NW * 128 * 8
    ep = ((n_edges + quantum - 1) // quantum) * quantum
    pad = ep - n_edges
    pad_iota = jnp.arange(pad, dtype=jnp.int32)
    rowp = jnp.concatenate(
        [row, pad_iota % N_NODES]).reshape(ep // 128, 128)
    colp = jnp.concatenate(
        [col, N_NODES + pad_iota % (NP - N_NODES)]).reshape(ep // 128, 128)

    zeros1d = jnp.zeros((10240,), jnp.float32)
    zeros128 = jnp.zeros((SLICE, FEAT), jnp.float32)
    b1r = b1.reshape(1, FEAT)
    b2r = b2.reshape(1, FEAT)
    bf1r = bf1.reshape(1, -1)
    bf2r = bf2.reshape(1, -1)
    batch3 = batch.reshape(GRID, 1, BLK)

    degp = _deg_call(colp, zeros1d)
    dp3 = (degp.reshape(NC, 10240)[:, :N_NODES]
           .reshape(NC, GRID, BLK).transpose(1, 0, 2))
    hs1 = _first_tc(dp3, x, W1)
    a1 = _agg_call(hs1, rowp, colp, zeros128)
    hs2 = _mid_tc(a1, hs1, dp3, b1r, W2)
    a2 = _agg_call(hs2, rowp, colp, zeros128)
    return _final_tc(a2, hs2, dp3, b2r, batch3, Wf1, bf1r, Wf2, bf2r)


# speedup vs baseline: 1.3013x; 1.0377x over previous
import functools

import jax
import jax.numpy as jnp
from jax import lax
from jax.experimental import pallas as pl
from jax.experimental.pallas import tpu as pltpu
from jax.experimental.pallas import tpu_sc as plsc

N_NODES = 10000
FEAT = 128
N_GRAPHS = 64

NC = 2
NT = 16
NW = NC * NT
NP = 10112
SLICE = NP // NT

BLK = 1000
GRID = N_NODES // BLK


def _sc_mesh():
    return plsc.VectorSubcoreMesh(core_axis_name="c", subcore_axis_name="s")


def _deg_call(coli, zeros1d):
    chunks = coli.shape[0]
    rpt = chunks // NW
    npd = 10240
    tile_rows = npd // NT

    @functools.partial(
        pl.kernel,
        out_type=jax.ShapeDtypeStruct((NC * npd,), jnp.float32),
        mesh=_sc_mesh(),
        scratch_types=[
            pltpu.VMEM((rpt, 128), jnp.int32),
            pltpu.VMEM((128,), jnp.float32),
            pltpu.VMEM_SHARED((npd,), jnp.float32),
            pltpu.SemaphoreType.DMA,
        ],
    )
    def deg_k(coli_hbm, zeros_hbm, out_hbm, colv, onesv, deg_sh, sem):
        c = lax.axis_index("c")
        s = lax.axis_index("s")
        wid = c * NT + s
        pltpu.sync_copy(zeros_hbm.at[pl.ds(s * tile_rows, tile_rows)],
                        deg_sh.at[pl.ds(s * tile_rows, tile_rows)])
        pltpu.sync_copy(coli_hbm.at[pl.ds(wid * rpt, rpt)], colv)
        for k in range(8):
            onesv[pl.ds(k * 16, 16)] = jnp.ones((16,), jnp.float32)
        plsc.subcore_barrier()

        def wait_one():
            pltpu.make_async_copy(zeros_hbm.at[pl.ds(0, 128)], onesv,
                                  sem).wait()

        pltpu.async_copy(onesv, deg_sh.at[colv.at[0]], sem, add=True)

        def body(j, carry):
            pltpu.async_copy(onesv, deg_sh.at[colv.at[j]], sem, add=True)
            wait_one()
            return carry

        lax.fori_loop(1, rpt, body, 0)
        wait_one()
        plsc.subcore_barrier()
        pltpu.sync_copy(deg_sh.at[pl.ds(s * tile_rows, tile_rows)],
                        out_hbm.at[pl.ds(c * npd + s * tile_rows, tile_rows)])

    return deg_k(coli, zeros1d)


def _agg_call(hs, rowi, coli, zeros128):
    chunks = rowi.shape[0]
    rpt = chunks // NW

    @functools.partial(
        pl.kernel,
        out_type=jax.ShapeDtypeStruct((NC, NP, FEAT), jnp.float32),
        mesh=_sc_mesh(),
        scratch_types=[
            pltpu.VMEM((40, 128), jnp.int32),
            pltpu.VMEM((40, 128), jnp.int32),
            pltpu.VMEM((128, FEAT), jnp.float32),
            pltpu.VMEM((128, FEAT), jnp.float32),
            pltpu.VMEM_SHARED((NP, FEAT), jnp.float32),
            pltpu.SemaphoreType.DMA,
        ],
    )
    def agg_k(hs_hbm, rowi_hbm, coli_hbm, zeros_hbm, out_hbm,
              rowv, colv, gbuf0, gbuf1, acc_sh, sem):
        c = lax.axis_index("c")
        s = lax.axis_index("s")
        wid = c * NT + s
        seg = 40
        nseg = rpt // seg
        pltpu.sync_copy(zeros_hbm, acc_sh.at[pl.ds(s * SLICE, SLICE)])
        plsc.subcore_barrier()

        def seg_body(g, carry):
            base = wid * rpt + g * seg
            pltpu.sync_copy(rowi_hbm.at[pl.ds(base, seg)], rowv)
            pltpu.sync_copy(coli_hbm.at[pl.ds(base, seg)], colv)
            pltpu.async_copy(hs_hbm.at[rowv.at[0]], gbuf0, sem)

            def body(t, c2):
                j0 = t * 2
                pltpu.make_async_copy(hs_hbm.at[rowv.at[0]], gbuf0, sem).wait()
                pltpu.async_copy(hs_hbm.at[rowv.at[j0 + 1]], gbuf1, sem)
                pltpu.sync_copy(gbuf0, acc_sh.at[colv.at[j0]], add=True)
                pltpu.make_async_copy(hs_hbm.at[rowv.at[0]], gbuf1, sem).wait()

                @pl.when(t + 1 < seg // 2)
                def _():
                    pltpu.async_copy(hs_hbm.at[rowv.at[j0 + 2]], gbuf0, sem)

                pltpu.sync_copy(gbuf1, acc_sh.at[colv.at[j0 + 1]], add=True)
                return c2

            lax.fori_loop(0, seg // 2, body, 0)
            return carry

        lax.fori_loop(0, nseg, seg_body, 0)
        plsc.subcore_barrier()
        pltpu.sync_copy(acc_sh.at[pl.ds(s * SLICE, SLICE)],
                        out_hbm.at[c, pl.ds(s * SLICE, SLICE)])

    return agg_k(hs, rowi, coli, zeros128)


def _dis_col(dp_ref):
    dvals = dp_ref[0]
    deg = dvals[0:1, :] + dvals[1:2, :] + 1.0
    return jnp.transpose(lax.rsqrt(deg), (1, 0))


def _first_tc(dp3, x, W1):
    def body(dp_ref, x_ref, w_ref, hs_ref):
        dis = _dis_col(dp_ref)
        h = jnp.dot(x_ref[...], w_ref[...], preferred_element_type=jnp.float32)
        hs_ref[...] = h * dis

    return pl.pallas_call(
        body,
        grid=(GRID,),
        in_specs=[
            pl.BlockSpec((1, NC, BLK), lambda i: (i, 0, 0)),
            pl.BlockSpec((BLK, FEAT), lambda i: (i, 0)),
            pl.BlockSpec((FEAT, FEAT), lambda i: (0, 0)),
        ],
        out_specs=pl.BlockSpec((BLK, FEAT), lambda i: (i, 0)),
        out_shape=jax.ShapeDtypeStruct((N_NODES, FEAT), jnp.float32),
    )(dp3, x, W1)


def _mid_tc(accp, hs, dp3, b, W):
    def body(accp_ref, hs_ref, dp_ref, b_ref, w_ref, out_ref):
        dis = _dis_col(dp_ref)
        agg = accp_ref[0] + accp_ref[1] + hs_ref[...]
        xn = jnp.maximum(dis * agg + b_ref[...], 0.0)
        h = jnp.dot(xn, w_ref[...], preferred_element_type=jnp.float32)
        out_ref[...] = h * dis

    return pl.pallas_call(
        body,
        grid=(GRID,),
        in_specs=[
            pl.BlockSpec((NC, BLK, FEAT), lambda i: (0, i, 0)),
            pl.BlockSpec((BLK, FEAT), lambda i: (i, 0)),
            pl.BlockSpec((1, NC, BLK), lambda i: (i, 0, 0)),
            pl.BlockSpec((1, FEAT), lambda i: (0, 0)),
            pl.BlockSpec((FEAT, FEAT), lambda i: (0, 0)),
        ],
        out_specs=pl.BlockSpec((BLK, FEAT), lambda i: (i, 0)),
        out_shape=jax.ShapeDtypeStruct((N_NODES, FEAT), jnp.float32),
    )(accp, hs, dp3, b, W)


def _final_tc(accp, hs, dp3, b, batch3, Wf1, bf1, Wf2, bf2):
    h3 = Wf1.shape[1]
    nout = Wf2.shape[1]

    def body(accp_ref, hs_ref, dp_ref, b_ref, batch_ref,
             wf1_ref, bf1_ref, wf2_ref, bf2_ref, out_ref, pooled, counts):
        i = pl.program_id(0)

        @pl.when(i == 0)
        def _():
            pooled[...] = jnp.zeros_like(pooled)
            counts[...] = jnp.zeros_like(counts)

        dis = _dis_col(dp_ref)
        agg = accp_ref[0] + accp_ref[1] + hs_ref[...]
        x3 = jnp.maximum(dis * agg + b_ref[...], 0.0)
        batch_row = batch_ref[0]
        giota = lax.broadcasted_iota(jnp.int32, (N_GRAPHS, BLK), 0)
        onehot_t = (giota == batch_row).astype(jnp.float32)
        pooled[...] += lax.dot_general(
            onehot_t, x3, (((1,), (0,)), ((), ())),
            preferred_element_type=jnp.float32)
        counts[...] += jnp.broadcast_to(
            jnp.sum(onehot_t, axis=1, keepdims=True), (N_GRAPHS, FEAT))

        @pl.when(i == GRID - 1)
        def _():
            mean = pooled[...] / jnp.maximum(counts[...], 1.0)
            hmid = jnp.dot(mean, wf1_ref[...],
                           preferred_element_type=jnp.float32) + bf1_ref[...]
            out_ref[...] = jnp.dot(hmid, wf2_ref[...],
                                   preferred_element_type=jnp.float32) + bf2_ref[...]

    return pl.pallas_call(
        body,
        grid=(GRID,),
        in_specs=[
            pl.BlockSpec((NC, BLK, FEAT), lambda i: (0, i, 0)),
            pl.BlockSpec((BLK, FEAT), lambda i: (i, 0)),
            pl.BlockSpec((1, NC, BLK), lambda i: (i, 0, 0)),
            pl.BlockSpec((1, FEAT), lambda i: (0, 0)),
            pl.BlockSpec((1, 1, BLK), lambda i: (i, 0, 0)),
            pl.BlockSpec((FEAT, h3), lambda i: (0, 0)),
            pl.BlockSpec((1, h3), lambda i: (0, 0)),
            pl.BlockSpec((h3, nout), lambda i: (0, 0)),
            pl.BlockSpec((1, nout), lambda i: (0, 0)),
        ],
        out_specs=pl.BlockSpec((N_GRAPHS, nout), lambda i: (0, 0)),
        out_shape=jax.ShapeDtypeStruct((N_GRAPHS, nout), jnp.float32),
        scratch_shapes=[
            pltpu.VMEM((N_GRAPHS, FEAT), jnp.float32),
            pltpu.VMEM((N_GRAPHS, FEAT), jnp.float32),
        ],
    )(accp, hs, dp3, b, batch3, Wf1, bf1, Wf2, bf2)


def kernel(x, edge_index, batch, W1, b1, W2, b2, Wf1, bf1, Wf2, bf2):
    row = edge_index[0]
    col = edge_index[1]
    n_edges = row.shape[0]
    quantum = NW * 128 * 8
    ep = ((n_edges + quantum - 1) // quantum) * quantum
    pad = ep - n_edges
    pad_iota = jnp.arange(pad, dtype=jnp.int32)
    rowp = jnp.concatenate(
        [row, pad_iota % N_NODES]).reshape(ep // 128, 128)
    colp = jnp.concatenate(
        [col, N_NODES + pad_iota % (NP - N_NODES)]).reshape(ep // 128, 128)

    zeros1d = jnp.zeros((10240,), jnp.float32)
    zeros128 = jnp.zeros((SLICE, FEAT), jnp.float32)
    b1r = b1.reshape(1, FEAT)
    b2r = b2.reshape(1, FEAT)
    bf1r = bf1.reshape(1, -1)
    bf2r = bf2.reshape(1, -1)
    batch3 = batch.reshape(GRID, 1, BLK)

    degp = _deg_call(colp, zeros1d)
    dp3 = (degp.reshape(NC, 10240)[:, :N_NODES]
           .reshape(NC, GRID, BLK).transpose(1, 0, 2))
    hs1 = _first_tc(dp3, x, W1)
    a1 = _agg_call(hs1, rowp, colp, zeros128)
    hs2 = _mid_tc(a1, hs1, dp3, b1r, W2)
    a2 = _agg_call(hs2, rowp, colp, zeros128)
    return _final_tc(a2, hs2, dp3, b2r, batch3, Wf1, bf1r, Wf2, bf2r)
